# baseline (speedup 1.0000x reference)
import functools

import jax
import jax.numpy as jnp
from jax import lax
from jax.experimental import pallas as pl
from jax.experimental.pallas import tpu as pltpu
from jax.experimental.pallas import tpu_sc as plsc

N_CNS = 2048
DV = 3
DC = 6
NUM_ITER = 20
LLR_MAX = 20.0


def _phi(x):
    x = jnp.clip(x, 8.5e-8, 16.635532)
    t = jnp.exp(x)
    return jnp.log((t + 1.0) / (t - 1.0))


def _bwd_vn_fwd(msg_c_flat, gc_chunks, gs_chunks, llr):
    n_vns, batch = llr.shape
    vpw = n_vns // _SC_NW
    spw = vpw * DV
    cpw = spw // 128
    mesh = plsc.VectorSubcoreMesh(core_axis_name="c", subcore_axis_name="s")

    @functools.partial(
        pl.kernel, mesh=mesh,
        out_type=[
            jax.ShapeDtypeStruct((DV * n_vns, batch), jnp.float32),
            jax.ShapeDtypeStruct((n_vns, batch), jnp.float32),
        ],
        scratch_types=[
            pltpu.VMEM((cpw, 128), jnp.int32),
            pltpu.VMEM((cpw, 128), jnp.int32),
            pltpu.VMEM((spw, batch), jnp.float32),
            pltpu.VMEM((vpw, batch), jnp.float32),
            pltpu.VMEM((spw, batch), jnp.float32),
            pltpu.VMEM((vpw, batch), jnp.float32),
            pltpu.SemaphoreType.DMA,
        ],
        compiler_params=pltpu.CompilerParams(use_tc_tiling_on_sc=False),
    )
    def bwd_vn_fwd_k(msgc_hbm, gc_hbm, gs_hbm, llr_hbm, mc_hbm, tot_hbm,
                     idx_v, idx2_v, rows_v, llr_v, out_v, tot_v, sem):
        wid = lax.axis_index("s") * _SC_NC + lax.axis_index("c")
        pltpu.sync_copy(gc_hbm.at[wid], idx_v)
        pltpu.sync_copy(gs_hbm.at[wid], idx2_v)
        pltpu.sync_copy(llr_hbm.at[pl.ds(wid * vpw, vpw)], llr_v)
        handles = [
            pltpu.async_copy(msgc_hbm.at[idx_v.at[i]],
                             rows_v.at[pl.ds(128 * i, 128)], sem)
            for i in range(cpw)
        ]
        for h in handles:
            h.wait()

        def body(vi, carry):
            base = vi * DV
            for t in range(batch // 16):
                sl = pl.ds(16 * t, 16)
                m0 = rows_v[base, sl]
                m1 = rows_v[base + 1, sl]
                m2 = rows_v[base + 2, sl]
                tt = llr_v[vi, sl] + m0 + m1 + m2
                tot_v[vi, sl] = tt
                out_v[base, sl] = tt - m0
                out_v[base + 1, sl] = tt - m1
                out_v[base + 2, sl] = tt - m2
            return carry

        lax.fori_loop(0, vpw, body, 0)
        scatters = [
            pltpu.async_copy(out_v.at[pl.ds(128 * i, 128)],
                             mc_hbm.at[idx2_v.at[i]], sem)
            for i in range(cpw)
        ]
        for h in scatters:
            h.wait()
        pltpu.sync_copy(tot_v, tot_hbm.at[pl.ds(wid * vpw, vpw)])

    return bwd_vn_fwd_k(msg_c_flat, gc_chunks, gs_chunks, llr)


def _cn_body(mc_ref, cmask_ref, out_ref):
    m = [mc_ref[k] for k in range(DC)]
    msk = [cmask_ref[k] for k in range(DC)]
    sgn = [jnp.where(msk[k] > 0.0,
                     jnp.where(m[k] < 0.0, -1.0, 1.0), 1.0) for k in range(DC)]
    mag = [jnp.where(msk[k] > 0.0,
                     _phi(jnp.clip(jnp.abs(m[k]), 0.0, LLR_MAX)), 0.0)
           for k in range(DC)]
    sign_node = sgn[0]
    mag_tot = mag[0]
    for k in range(1, DC):
        sign_node = sign_node * sgn[k]
        mag_tot = mag_tot + mag[k]
    for k in range(DC):
        out_ref[k] = (sign_node * sgn[k]) * _phi(mag_tot - mag[k])
    out_ref[DC] = jnp.zeros_like(out_ref[DC])


def _cn_update(mc, cmask, *, c_blk=256):
    _, n_cns, batch = mc.shape
    grid = (n_cns // c_blk,)
    return pl.pallas_call(
        _cn_body,
        grid=grid,
        in_specs=[
            pl.BlockSpec((DC, c_blk, batch), lambda i: (0, i, 0)),
            pl.BlockSpec((DC, c_blk, 1), lambda i: (0, i, 0)),
        ],
        out_specs=pl.BlockSpec((DC + 1, c_blk, batch), lambda i: (0, i, 0)),
        out_shape=jax.ShapeDtypeStruct((DC + 1, n_cns, batch), jnp.float32),
    )(mc, cmask)


_SC_NC = 2
_SC_NS = 16
_SC_NW = _SC_NC * _SC_NS


def _row_gather(src_flat, idx_chunks):
    nw, cpw, _ = idx_chunks.shape
    batch = src_flat.shape[1]
    mesh = plsc.VectorSubcoreMesh(core_axis_name="c", subcore_axis_name="s")

    @functools.partial(
        pl.kernel, mesh=mesh,
        out_type=jax.ShapeDtypeStruct((nw * cpw, 128, batch), jnp.float32),
        scratch_types=[
            pltpu.VMEM((cpw, 128), jnp.int32),
            pltpu.VMEM((cpw, 128, batch), jnp.float32),
            pltpu.SemaphoreType.DMA,
        ],
        compiler_params=pltpu.CompilerParams(use_tc_tiling_on_sc=False),
    )
    def gather_k(src_hbm, idx_hbm, out_hbm, idx_v, rows_v, sem):
        wid = lax.axis_index("s") * _SC_NC + lax.axis_index("c")
        pltpu.sync_copy(idx_hbm.at[wid], idx_v)
        handles = [
            pltpu.async_copy(src_hbm.at[idx_v.at[i]], rows_v.at[i], sem)
            for i in range(cpw)
        ]
        for h in handles:
            h.wait()
        pltpu.sync_copy(rows_v, out_hbm.at[pl.ds(wid * cpw, cpw)])

    return gather_k(src_flat, idx_chunks)


def _seg_slot(ids, depth):
    slot = jnp.zeros(ids.shape, jnp.int32)
    run = jnp.ones(ids.shape, jnp.bool_)
    for t in range(1, depth):
        sh = jnp.concatenate([jnp.full((t,), -1, ids.dtype), ids[:-t]])
        run = run & (ids == sh)
        slot = slot + run.astype(jnp.int32)
    return slot


def _seg_starts(ids, n_segs, num_edges):
    targets = jnp.arange(n_segs + 1, dtype=jnp.int32)
    return jnp.sum(ids.astype(jnp.int32)[None, :] < targets[:, None],
                   axis=1, dtype=jnp.int32)


def _setup(vn_con, cn_ids, ind_cn, ind_cn_inv, n_vns):
    num_edges = vn_con.shape[0]

    j_slot = _seg_slot(vn_con, DV)
    k_slot = _seg_slot(cn_ids, DC)
    cs = k_slot * N_CNS + cn_ids.astype(jnp.int32)

    vstart = _seg_starts(vn_con, n_vns, num_edges)
    cstart = _seg_starts(cn_ids, N_CNS, num_edges)
    deg_v = vstart[1:] - vstart[:-1]
    deg_c = cstart[1:] - cstart[:-1]
    vmask = (jnp.arange(DV, dtype=jnp.int32)[None, :] < deg_v[:, None])
    cmask = (jnp.arange(DC, dtype=jnp.int32)[:, None] < deg_c[None, :])

    e_of_s = jnp.minimum(vstart[:-1][:, None]
                         + jnp.arange(DV, dtype=jnp.int32)[None, :],
                         num_edges - 1)

    base = jnp.take(jnp.take(cs, ind_cn_inv), e_of_s.reshape(-1))
    vmask_flat = vmask.reshape(-1)
    cmask_f = cmask.astype(jnp.float32)
    dummy = jnp.argmin(cmask_f.reshape(-1)).astype(jnp.int32)
    gc = jnp.where(vmask_flat, base, DC * N_CNS)
    gs = jnp.where(vmask_flat, base, dummy)
    return gc, gs, cmask_f.reshape(DC, N_CNS, 1)


def kernel(llr_ch, vn_con, cn_ids, ind_cn, ind_cn_inv):
    batch, n_vns = llr_ch.shape
    llr = -1.0 * jnp.transpose(llr_ch.astype(jnp.float32))
    gc, gs, cmask = _setup(vn_con, cn_ids, ind_cn, ind_cn_inv, n_vns)

    gc_chunks = gc.reshape(_SC_NW, -1, 128)
    gs_chunks = gs.reshape(_SC_NW, -1, 128)

    hb = batch // 2
    llr_h = [llr[:, :hb], llr[:, hb:]]
    msg_c = [jnp.zeros(((DC + 1) * N_CNS, hb), jnp.float32) for _ in range(2)]
    for _ in range(NUM_ITER):
        mc = [None, None]
        for h in range(2):
            mc[h], _ = _bwd_vn_fwd(msg_c[h], gc_chunks, gs_chunks, llr_h[h])
        for h in range(2):
            msg_c[h] = _cn_update(mc[h].reshape(DC, N_CNS, hb),
                                  cmask).reshape((DC + 1) * N_CNS, hb)
    tot = [None, None]
    for h in range(2):
        _, tot[h] = _bwd_vn_fwd(msg_c[h], gc_chunks, gs_chunks, llr_h[h])
    return -1.0 * jnp.transpose(jnp.concatenate(tot, axis=1))


# speedup vs baseline: 2.2213x; 2.2213x over previous
import functools

import jax
import jax.numpy as jnp
from jax import lax
from jax.experimental import pallas as pl
from jax.experimental.pallas import tpu as pltpu
from jax.experimental.pallas import tpu_sc as plsc

N_CNS = 2048
DV = 3
DC = 6
NUM_ITER = 20
LLR_MAX = 20.0


def _phi(x):
    x = jnp.clip(x, 8.5e-8, 16.635532)
    t = jnp.exp(x)
    return jnp.log((t + 1.0) / (t - 1.0))


def _bwd_vn_fwd(msg_c_flat, gc_chunks, gs_chunks, llr):
    n_vns, batch = llr.shape
    vpw = n_vns // _SC_NW
    spw = vpw * DV
    cpw = spw // 128
    mesh = plsc.VectorSubcoreMesh(core_axis_name="c", subcore_axis_name="s")

    @functools.partial(
        pl.kernel, mesh=mesh,
        out_type=[
            jax.ShapeDtypeStruct((DV * n_vns, batch), jnp.float32),
            jax.ShapeDtypeStruct((n_vns, batch), jnp.float32),
        ],
        scratch_types=[
            pltpu.VMEM((cpw, 128), jnp.int32),
            pltpu.VMEM((cpw, 128), jnp.int32),
            pltpu.VMEM((spw, batch), jnp.float32),
            pltpu.VMEM((vpw, batch), jnp.float32),
            pltpu.VMEM((spw, batch), jnp.float32),
            pltpu.VMEM((vpw, batch), jnp.float32),
            pltpu.SemaphoreType.DMA,
        ],
        compiler_params=pltpu.CompilerParams(use_tc_tiling_on_sc=False),
    )
    def bwd_vn_fwd_k(msgc_hbm, gc_hbm, gs_hbm, llr_hbm, mc_hbm, tot_hbm,
                     idx_v, idx2_v, rows_v, llr_v, out_v, tot_v, sem):
        wid = lax.axis_index("s") * _SC_NC + lax.axis_index("c")
        pltpu.sync_copy(gc_hbm.at[wid], idx_v)
        handles = [
            pltpu.async_copy(msgc_hbm.at[idx_v.at[i]],
                             rows_v.at[pl.ds(128 * i, 128)], sem)
            for i in range(cpw)
        ]
        pltpu.sync_copy(gs_hbm.at[wid], idx2_v)
        pltpu.sync_copy(llr_hbm.at[pl.ds(wid * vpw, vpw)], llr_v)
        for h in handles:
            h.wait()

        def body(vi, carry):
            base = vi * DV
            for t in range(batch // 16):
                sl = pl.ds(16 * t, 16)
                m0 = rows_v[base, sl]
                m1 = rows_v[base + 1, sl]
                m2 = rows_v[base + 2, sl]
                tt = llr_v[vi, sl] + m0 + m1 + m2
                tot_v[vi, sl] = tt
                out_v[base, sl] = tt - m0
                out_v[base + 1, sl] = tt - m1
                out_v[base + 2, sl] = tt - m2
            return carry

        lax.fori_loop(0, vpw, body, 0)
        scatters = [
            pltpu.async_copy(out_v.at[pl.ds(128 * i, 128)],
                             mc_hbm.at[idx2_v.at[i]], sem)
            for i in range(cpw)
        ]
        for h in scatters:
            h.wait()
        pltpu.sync_copy(tot_v, tot_hbm.at[pl.ds(wid * vpw, vpw)])

    return bwd_vn_fwd_k(msg_c_flat, gc_chunks, gs_chunks, llr)


def _cn_body(mc_ref, cmask_ref, out_ref):
    m = [mc_ref[k] for k in range(DC)]
    msk = [cmask_ref[k] for k in range(DC)]
    sgn = [jnp.where(msk[k] > 0.0,
                     jnp.where(m[k] < 0.0, -1.0, 1.0), 1.0) for k in range(DC)]
    mag = [jnp.where(msk[k] > 0.0,
                     _phi(jnp.clip(jnp.abs(m[k]), 0.0, LLR_MAX)), 0.0)
           for k in range(DC)]
    sign_node = sgn[0]
    mag_tot = mag[0]
    for k in range(1, DC):
        sign_node = sign_node * sgn[k]
        mag_tot = mag_tot + mag[k]
    for k in range(DC):
        out_ref[k] = (sign_node * sgn[k]) * _phi(mag_tot - mag[k])
    out_ref[DC] = jnp.zeros_like(out_ref[DC])


def _cn_update(mc, cmask_wide, *, c_blk=128):
    _, n_rows, width = mc.shape
    grid = (n_rows // c_blk,)
    return pl.pallas_call(
        _cn_body,
        grid=grid,
        in_specs=[
            pl.BlockSpec((DC, c_blk, width), lambda i: (0, i, 0)),
            pl.BlockSpec((DC, c_blk, width), lambda i: (0, i, 0)),
        ],
        out_specs=pl.BlockSpec((DC + 1, c_blk, width), lambda i: (0, i, 0)),
        out_shape=jax.ShapeDtypeStruct((DC + 1, n_rows, width), jnp.float32),
    )(mc, cmask_wide)


_SC_NC = 2
_SC_NS = 16
_SC_NW = _SC_NC * _SC_NS


def _row_gather(src_flat, idx_chunks):
    nw, cpw, _ = idx_chunks.shape
    batch = src_flat.shape[1]
    mesh = plsc.VectorSubcoreMesh(core_axis_name="c", subcore_axis_name="s")

    @functools.partial(
        pl.kernel, mesh=mesh,
        out_type=jax.ShapeDtypeStruct((nw * cpw, 128, batch), jnp.float32),
        scratch_types=[
            pltpu.VMEM((cpw, 128), jnp.int32),
            pltpu.VMEM((cpw, 128, batch), jnp.float32),
            pltpu.SemaphoreType.DMA,
        ],
        compiler_params=pltpu.CompilerParams(use_tc_tiling_on_sc=False),
    )
    def gather_k(src_hbm, idx_hbm, out_hbm, idx_v, rows_v, sem):
        wid = lax.axis_index("s") * _SC_NC + lax.axis_index("c")
        pltpu.sync_copy(idx_hbm.at[wid], idx_v)
        handles = [
            pltpu.async_copy(src_hbm.at[idx_v.at[i]], rows_v.at[i], sem)
            for i in range(cpw)
        ]
        for h in handles:
            h.wait()
        pltpu.sync_copy(rows_v, out_hbm.at[pl.ds(wid * cpw, cpw)])

    return gather_k(src_flat, idx_chunks)


def _seg_slot(ids, depth):
    slot = jnp.zeros(ids.shape, jnp.int32)
    run = jnp.ones(ids.shape, jnp.bool_)
    for t in range(1, depth):
        sh = jnp.concatenate([jnp.full((t,), -1, ids.dtype), ids[:-t]])
        run = run & (ids == sh)
        slot = slot + run.astype(jnp.int32)
    return slot


def _seg_starts(ids, n_segs, num_edges):
    targets = jnp.arange(n_segs + 1, dtype=jnp.int32)
    return jnp.sum(ids.astype(jnp.int32)[None, :] < targets[:, None],
                   axis=1, dtype=jnp.int32)


def _setup(vn_con, cn_ids, ind_cn, ind_cn_inv, n_vns):
    num_edges = vn_con.shape[0]

    j_slot = _seg_slot(vn_con, DV)
    k_slot = _seg_slot(cn_ids, DC)
    cs = k_slot * N_CNS + cn_ids.astype(jnp.int32)

    vstart = _seg_starts(vn_con, n_vns, num_edges)
    cstart = _seg_starts(cn_ids, N_CNS, num_edges)
    deg_v = vstart[1:] - vstart[:-1]
    deg_c = cstart[1:] - cstart[:-1]
    vmask = (jnp.arange(DV, dtype=jnp.int32)[None, :] < deg_v[:, None])
    cmask = (jnp.arange(DC, dtype=jnp.int32)[:, None] < deg_c[None, :])

    e_of_s = jnp.minimum(vstart[:-1][:, None]
                         + jnp.arange(DV, dtype=jnp.int32)[None, :],
                         num_edges - 1)

    base = jnp.take(jnp.take(cs, ind_cn_inv), e_of_s.reshape(-1))
    vmask_flat = vmask.reshape(-1)
    cmask_f = cmask.astype(jnp.float32)
    dummy = jnp.argmin(cmask_f.reshape(-1)).astype(jnp.int32)
    gc = jnp.where(vmask_flat, base, DC * N_CNS)
    gs = jnp.where(vmask_flat, base, dummy)
    return gc, gs, cmask_f


def kernel(llr_ch, vn_con, cn_ids, ind_cn, ind_cn_inv):
    batch, n_vns = llr_ch.shape
    llr = -1.0 * jnp.transpose(llr_ch.astype(jnp.float32))
    gc, gs, cmask_f = _setup(vn_con, cn_ids, ind_cn, ind_cn_inv, n_vns)

    gc_chunks = gc.reshape(_SC_NW, -1, 128)
    gs_chunks = gs.reshape(_SC_NW, -1, 128)
    n_rows = N_CNS * batch // 128
    cmask_wide = jnp.broadcast_to(cmask_f[:, :, None],
                                  (DC, N_CNS, batch)).reshape(DC, n_rows, 128)

    msg_c = jnp.zeros(((DC + 1) * N_CNS, batch), jnp.float32)
    for _ in range(NUM_ITER):
        mc, _tot = _bwd_vn_fwd(msg_c, gc_chunks, gs_chunks, llr)
        msg_c = _cn_update(mc.reshape(DC, n_rows, 128),
                           cmask_wide).reshape((DC + 1) * N_CNS, batch)
    _, tot = _bwd_vn_fwd(msg_c, gc_chunks, gs_chunks, llr)
    return -1.0 * jnp.transpose(tot)


# speedup vs baseline: 2.4187x; 1.0889x over previous
import functools

import jax
import jax.numpy as jnp
from jax import lax
from jax.experimental import pallas as pl
from jax.experimental.pallas import tpu as pltpu
from jax.experimental.pallas import tpu_sc as plsc

N_CNS = 2048
DV = 3
DC = 6
NUM_ITER = 20
LLR_MAX = 20.0


def _phi(x):
    x = jnp.clip(x, 8.5e-8, 16.635532)
    t = jnp.exp(x)
    return jnp.log((t + 1.0) / (t - 1.0))


def _bwd_vn_fwd(msg_c_flat, gc_chunks, gs_chunks, llr):
    n_vns, batch = llr.shape
    vpw = n_vns // _SC_NW
    spw = vpw * DV
    cpw = spw // 128
    mesh = plsc.VectorSubcoreMesh(core_axis_name="c", subcore_axis_name="s")

    @functools.partial(
        pl.kernel, mesh=mesh,
        out_type=[
            jax.ShapeDtypeStruct((DV * n_vns, batch), jnp.float32),
            jax.ShapeDtypeStruct((n_vns, batch), jnp.float32),
        ],
        scratch_types=[
            pltpu.VMEM((cpw, 128), jnp.int32),
            pltpu.VMEM((cpw, 128), jnp.int32),
            pltpu.VMEM((spw, batch), jnp.float32),
            pltpu.VMEM((vpw, batch), jnp.float32),
            pltpu.VMEM((spw, batch), jnp.float32),
            pltpu.VMEM((vpw, batch), jnp.float32),
            pltpu.SemaphoreType.DMA,
        ],
        compiler_params=pltpu.CompilerParams(use_tc_tiling_on_sc=False),
    )
    def bwd_vn_fwd_k(msgc_hbm, gc_hbm, gs_hbm, llr_hbm, mc_hbm, tot_hbm,
                     idx_v, idx2_v, rows_v, llr_v, out_v, tot_v, sem):
        wid = lax.axis_index("s") * _SC_NC + lax.axis_index("c")
        pltpu.sync_copy(gc_hbm.at[wid], idx_v)
        handles = [
            pltpu.async_copy(msgc_hbm.at[idx_v.at[i]],
                             rows_v.at[pl.ds(128 * i, 128)], sem)
            for i in range(cpw)
        ]
        pltpu.sync_copy(gs_hbm.at[wid], idx2_v)
        pltpu.sync_copy(llr_hbm.at[pl.ds(wid * vpw, vpw)], llr_v)
        for h in handles:
            h.wait()

        def body(vi, carry):
            base = vi * DV
            for t in range(batch // 16):
                sl = pl.ds(16 * t, 16)
                m0 = rows_v[base, sl]
                m1 = rows_v[base + 1, sl]
                m2 = rows_v[base + 2, sl]
                tt = llr_v[vi, sl] + m0 + m1 + m2
                tot_v[vi, sl] = tt
                out_v[base, sl] = tt - m0
                out_v[base + 1, sl] = tt - m1
                out_v[base + 2, sl] = tt - m2
            return carry

        lax.fori_loop(0, vpw, body, 0)
        scatters = [
            pltpu.async_copy(out_v.at[pl.ds(128 * i, 128)],
                             mc_hbm.at[idx2_v.at[i]], sem)
            for i in range(cpw)
        ]
        for h in scatters:
            h.wait()
        pltpu.sync_copy(tot_v, tot_hbm.at[pl.ds(wid * vpw, vpw)])

    return bwd_vn_fwd_k(msg_c_flat, gc_chunks, gs_chunks, llr)


def _cn_body(mc_ref, cmask_ref, out_ref):
    m = [mc_ref[k] for k in range(DC)]
    msk = [cmask_ref[k] for k in range(DC)]
    sgn = [jnp.where(msk[k] > 0.0,
                     jnp.where(m[k] < 0.0, -1.0, 1.0), 1.0) for k in range(DC)]
    mag = [jnp.where(msk[k] > 0.0,
                     _phi(jnp.clip(jnp.abs(m[k]), 0.0, LLR_MAX)), 0.0)
           for k in range(DC)]
    sign_node = sgn[0]
    mag_tot = mag[0]
    for k in range(1, DC):
        sign_node = sign_node * sgn[k]
        mag_tot = mag_tot + mag[k]
    for k in range(DC):
        out_ref[k] = (sign_node * sgn[k]) * _phi(mag_tot - mag[k])
    out_ref[DC] = jnp.zeros_like(out_ref[DC])


def _cn_update(mc, cmask_wide, *, c_blk=512):
    _, n_rows, width = mc.shape
    grid = (n_rows // c_blk,)
    return pl.pallas_call(
        _cn_body,
        grid=grid,
        in_specs=[
            pl.BlockSpec((DC, c_blk, width), lambda i: (0, i, 0)),
            pl.BlockSpec((DC, c_blk, width), lambda i: (0, i, 0)),
        ],
        out_specs=pl.BlockSpec((DC + 1, c_blk, width), lambda i: (0, i, 0)),
        out_shape=jax.ShapeDtypeStruct((DC + 1, n_rows, width), jnp.float32),
    )(mc, cmask_wide)


_SC_NC = 2
_SC_NS = 16
_SC_NW = _SC_NC * _SC_NS


def _row_gather(src_flat, idx_chunks):
    nw, cpw, _ = idx_chunks.shape
    batch = src_flat.shape[1]
    mesh = plsc.VectorSubcoreMesh(core_axis_name="c", subcore_axis_name="s")

    @functools.partial(
        pl.kernel, mesh=mesh,
        out_type=jax.ShapeDtypeStruct((nw * cpw, 128, batch), jnp.float32),
        scratch_types=[
            pltpu.VMEM((cpw, 128), jnp.int32),
            pltpu.VMEM((cpw, 128, batch), jnp.float32),
            pltpu.SemaphoreType.DMA,
        ],
        compiler_params=pltpu.CompilerParams(use_tc_tiling_on_sc=False),
    )
    def gather_k(src_hbm, idx_hbm, out_hbm, idx_v, rows_v, sem):
        wid = lax.axis_index("s") * _SC_NC + lax.axis_index("c")
        pltpu.sync_copy(idx_hbm.at[wid], idx_v)
        handles = [
            pltpu.async_copy(src_hbm.at[idx_v.at[i]], rows_v.at[i], sem)
            for i in range(cpw)
        ]
        for h in handles:
            h.wait()
        pltpu.sync_copy(rows_v, out_hbm.at[pl.ds(wid * cpw, cpw)])

    return gather_k(src_flat, idx_chunks)


def _seg_slot(ids, depth):
    slot = jnp.zeros(ids.shape, jnp.int32)
    run = jnp.ones(ids.shape, jnp.bool_)
    for t in range(1, depth):
        sh = jnp.concatenate([jnp.full((t,), -1, ids.dtype), ids[:-t]])
        run = run & (ids == sh)
        slot = slot + run.astype(jnp.int32)
    return slot


def _seg_starts(ids, n_segs, num_edges):
    targets = jnp.arange(n_segs + 1, dtype=jnp.int32)
    return jnp.sum(ids.astype(jnp.int32)[None, :] < targets[:, None],
                   axis=1, dtype=jnp.int32)


def _setup(vn_con, cn_ids, ind_cn, ind_cn_inv, n_vns):
    num_edges = vn_con.shape[0]

    j_slot = _seg_slot(vn_con, DV)
    k_slot = _seg_slot(cn_ids, DC)
    cs = k_slot * N_CNS + cn_ids.astype(jnp.int32)

    vstart = _seg_starts(vn_con, n_vns, num_edges)
    cstart = _seg_starts(cn_ids, N_CNS, num_edges)
    deg_v = vstart[1:] - vstart[:-1]
    deg_c = cstart[1:] - cstart[:-1]
    vmask = (jnp.arange(DV, dtype=jnp.int32)[None, :] < deg_v[:, None])
    cmask = (jnp.arange(DC, dtype=jnp.int32)[:, None] < deg_c[None, :])

    e_of_s = jnp.minimum(vstart[:-1][:, None]
                         + jnp.arange(DV, dtype=jnp.int32)[None, :],
                         num_edges - 1)

    base = jnp.take(jnp.take(cs, ind_cn_inv), e_of_s.reshape(-1))
    vmask_flat = vmask.reshape(-1)
    cmask_f = cmask.astype(jnp.float32)
    dummy = jnp.argmin(cmask_f.reshape(-1)).astype(jnp.int32)
    gc = jnp.where(vmask_flat, base, DC * N_CNS)
    gs = jnp.where(vmask_flat, base, dummy)
    return gc, gs, cmask_f


def kernel(llr_ch, vn_con, cn_ids, ind_cn, ind_cn_inv):
    batch, n_vns = llr_ch.shape
    llr = -1.0 * jnp.transpose(llr_ch.astype(jnp.float32))
    gc, gs, cmask_f = _setup(vn_con, cn_ids, ind_cn, ind_cn_inv, n_vns)

    gc_chunks = gc.reshape(_SC_NW, -1, 128)
    gs_chunks = gs.reshape(_SC_NW, -1, 128)
    n_rows = N_CNS * batch // 128
    cmask_wide = jnp.broadcast_to(cmask_f[:, :, None],
                                  (DC, N_CNS, batch)).reshape(DC, n_rows, 128)

    msg_c = jnp.zeros(((DC + 1) * N_CNS, batch), jnp.float32)
    for _ in range(NUM_ITER):
        mc, _tot = _bwd_vn_fwd(msg_c, gc_chunks, gs_chunks, llr)
        msg_c = _cn_update(mc.reshape(DC, n_rows, 128),
                           cmask_wide).reshape((DC + 1) * N_CNS, batch)
    _, tot = _bwd_vn_fwd(msg_c, gc_chunks, gs_chunks, llr)
    return -1.0 * jnp.transpose(tot)


# speedup vs baseline: 2.5949x; 1.0729x over previous
import functools

import jax
import jax.numpy as jnp
from jax import lax
from jax.experimental import pallas as pl
from jax.experimental.pallas import tpu as pltpu
from jax.experimental.pallas import tpu_sc as plsc

N_CNS = 2048
DV = 3
DC = 6
NUM_ITER = 20
LLR_MAX = 20.0


def _phi(x):
    x = jnp.clip(x, 8.5e-8, 16.635532)
    t = jnp.exp(x)
    return jnp.log((t + 1.0) / (t - 1.0))


def _bwd_vn_fwd(msg_c_flat, gc_chunks, gs_chunks, llr):
    n_vns, batch = llr.shape
    vpw = n_vns // _SC_NW
    spw = vpw * DV
    cpw = spw // 128
    mesh = plsc.VectorSubcoreMesh(core_axis_name="c", subcore_axis_name="s")

    @functools.partial(
        pl.kernel, mesh=mesh,
        out_type=[
            jax.ShapeDtypeStruct((DV * n_vns, batch), jnp.float32),
            jax.ShapeDtypeStruct((n_vns, batch), jnp.float32),
        ],
        scratch_types=[
            pltpu.VMEM((cpw, 128), jnp.int32),
            pltpu.VMEM((cpw, 128), jnp.int32),
            pltpu.VMEM((spw, batch), jnp.float32),
            pltpu.VMEM((vpw, batch), jnp.float32),
            pltpu.VMEM((spw, batch), jnp.float32),
            pltpu.VMEM((vpw, batch), jnp.float32),
            pltpu.SemaphoreType.DMA,
        ],
        compiler_params=pltpu.CompilerParams(use_tc_tiling_on_sc=False),
    )
    def bwd_vn_fwd_k(msgc_hbm, gc_hbm, gs_hbm, llr_hbm, mc_hbm, tot_hbm,
                     idx_v, idx2_v, rows_v, llr_v, out_v, tot_v, sem):
        wid = lax.axis_index("s") * _SC_NC + lax.axis_index("c")
        pltpu.sync_copy(gc_hbm.at[wid], idx_v)
        handles = [
            pltpu.async_copy(msgc_hbm.at[idx_v.at[i]],
                             rows_v.at[pl.ds(128 * i, 128)], sem)
            for i in range(cpw)
        ]
        pltpu.sync_copy(gs_hbm.at[wid], idx2_v)
        pltpu.sync_copy(llr_hbm.at[pl.ds(wid * vpw, vpw)], llr_v)
        for h in handles:
            h.wait()

        def body(vi, carry):
            base = vi * DV
            for t in range(batch // 16):
                sl = pl.ds(16 * t, 16)
                m0 = rows_v[base, sl]
                m1 = rows_v[base + 1, sl]
                m2 = rows_v[base + 2, sl]
                tt = llr_v[vi, sl] + m0 + m1 + m2
                tot_v[vi, sl] = tt
                out_v[base, sl] = tt - m0
                out_v[base + 1, sl] = tt - m1
                out_v[base + 2, sl] = tt - m2
            return carry

        lax.fori_loop(0, vpw, body, 0)
        scatters = [
            pltpu.async_copy(out_v.at[pl.ds(128 * i, 128)],
                             mc_hbm.at[idx2_v.at[i]], sem)
            for i in range(cpw)
        ]
        for h in scatters:
            h.wait()
        pltpu.sync_copy(tot_v, tot_hbm.at[pl.ds(wid * vpw, vpw)])

    return bwd_vn_fwd_k(msg_c_flat, gc_chunks, gs_chunks, llr)


def _cn_body(mc_ref, cmask_ref, out_ref):
    m = [mc_ref[k] for k in range(DC)]
    msk = [cmask_ref[k] for k in range(DC)]
    sgn = [jnp.where(msk[k] > 0.0,
                     jnp.where(m[k] < 0.0, -1.0, 1.0), 1.0) for k in range(DC)]
    mag = [jnp.where(msk[k] > 0.0,
                     _phi(jnp.clip(jnp.abs(m[k]), 0.0, LLR_MAX)), 0.0)
           for k in range(DC)]
    sign_node = sgn[0]
    mag_tot = mag[0]
    for k in range(1, DC):
        sign_node = sign_node * sgn[k]
        mag_tot = mag_tot + mag[k]
    for k in range(DC):
        out_ref[k] = (sign_node * sgn[k]) * _phi(mag_tot - mag[k])
    out_ref[DC] = jnp.zeros_like(out_ref[DC])


def _cn_update(mc, cmask_wide, *, c_blk=512):
    _, n_rows, width = mc.shape
    grid = (n_rows // c_blk,)
    return pl.pallas_call(
        _cn_body,
        grid=grid,
        in_specs=[
            pl.BlockSpec((DC, c_blk, width), lambda i: (0, i, 0)),
            pl.BlockSpec((DC, c_blk, width), lambda i: (0, i, 0)),
        ],
        out_specs=pl.BlockSpec((DC + 1, c_blk, width), lambda i: (0, i, 0)),
        out_shape=jax.ShapeDtypeStruct((DC + 1, n_rows, width), jnp.float32),
    )(mc, cmask_wide)


_SC_NC = 2
_SC_NS = 16
_SC_NW = _SC_NC * _SC_NS


def _row_gather(src_flat, idx_chunks):
    nw, cpw, _ = idx_chunks.shape
    batch = src_flat.shape[1]
    mesh = plsc.VectorSubcoreMesh(core_axis_name="c", subcore_axis_name="s")

    @functools.partial(
        pl.kernel, mesh=mesh,
        out_type=jax.ShapeDtypeStruct((nw * cpw, 128, batch), jnp.float32),
        scratch_types=[
            pltpu.VMEM((cpw, 128), jnp.int32),
            pltpu.VMEM((cpw, 128, batch), jnp.float32),
            pltpu.SemaphoreType.DMA,
        ],
        compiler_params=pltpu.CompilerParams(use_tc_tiling_on_sc=False),
    )
    def gather_k(src_hbm, idx_hbm, out_hbm, idx_v, rows_v, sem):
        wid = lax.axis_index("s") * _SC_NC + lax.axis_index("c")
        pltpu.sync_copy(idx_hbm.at[wid], idx_v)
        handles = [
            pltpu.async_copy(src_hbm.at[idx_v.at[i]], rows_v.at[i], sem)
            for i in range(cpw)
        ]
        for h in handles:
            h.wait()
        pltpu.sync_copy(rows_v, out_hbm.at[pl.ds(wid * cpw, cpw)])

    return gather_k(src_flat, idx_chunks)


def _seg_slot(ids, depth):
    slot = jnp.zeros(ids.shape, jnp.int32)
    run = jnp.ones(ids.shape, jnp.bool_)
    for t in range(1, depth):
        sh = jnp.concatenate([jnp.full((t,), -1, ids.dtype), ids[:-t]])
        run = run & (ids == sh)
        slot = slot + run.astype(jnp.int32)
    return slot


def _seg_starts(ids, n_segs, num_edges):
    targets = jnp.arange(n_segs + 1, dtype=jnp.int32)
    return jnp.sum(ids.astype(jnp.int32)[None, :] < targets[:, None],
                   axis=1, dtype=jnp.int32)


def _index_compose(cs, ind_cn_inv, e_of_s_flat, n_slots):
    num_edges = cs.shape[0]
    spw = n_slots // _SC_NW
    mesh = plsc.VectorSubcoreMesh(core_axis_name="c", subcore_axis_name="s")

    @functools.partial(
        pl.kernel, mesh=mesh,
        out_type=jax.ShapeDtypeStruct((_SC_NW, spw), jnp.int32),
        scratch_types=[
            pltpu.VMEM((num_edges,), jnp.int32),
            pltpu.VMEM((num_edges,), jnp.int32),
            pltpu.VMEM((spw,), jnp.int32),
            pltpu.VMEM((spw,), jnp.int32),
        ],
        compiler_params=pltpu.CompilerParams(use_tc_tiling_on_sc=False,
                                             needs_layout_passes=False),
    )
    def compose_k(cs_hbm, inv_hbm, eos_hbm, out_hbm, cs_v, inv_v, e_v, o_v):
        wid = lax.axis_index("s") * _SC_NC + lax.axis_index("c")
        pltpu.sync_copy(cs_hbm, cs_v)
        pltpu.sync_copy(inv_hbm, inv_v)
        pltpu.sync_copy(eos_hbm.at[wid], e_v)

        def body(i, carry):
            sl = pl.ds(16 * i, 16)
            ev = e_v[sl]
            p = plsc.load_gather(inv_v, [ev])
            o_v[sl] = plsc.load_gather(cs_v, [p])
            return carry

        lax.fori_loop(0, spw // 16, body, 0)
        pltpu.sync_copy(o_v, out_hbm.at[wid])

    return compose_k(cs, ind_cn_inv,
                     e_of_s_flat.reshape(_SC_NW, spw)).reshape(-1)


def _setup(vn_con, cn_ids, ind_cn, ind_cn_inv, n_vns):
    num_edges = vn_con.shape[0]

    j_slot = _seg_slot(vn_con, DV)
    k_slot = _seg_slot(cn_ids, DC)
    cs = k_slot * N_CNS + cn_ids.astype(jnp.int32)

    vstart = _seg_starts(vn_con, n_vns, num_edges)
    cstart = _seg_starts(cn_ids, N_CNS, num_edges)
    deg_v = vstart[1:] - vstart[:-1]
    deg_c = cstart[1:] - cstart[:-1]
    vmask = (jnp.arange(DV, dtype=jnp.int32)[None, :] < deg_v[:, None])
    cmask = (jnp.arange(DC, dtype=jnp.int32)[:, None] < deg_c[None, :])

    e_of_s = jnp.minimum(vstart[:-1][:, None]
                         + jnp.arange(DV, dtype=jnp.int32)[None, :],
                         num_edges - 1)

    base = _index_compose(cs, ind_cn_inv.astype(jnp.int32),
                          e_of_s.reshape(-1), DV * n_vns)
    vmask_flat = vmask.reshape(-1)
    cmask_f = cmask.astype(jnp.float32)
    dummy = jnp.argmin(cmask_f.reshape(-1)).astype(jnp.int32)
    gc = jnp.where(vmask_flat, base, DC * N_CNS)
    gs = jnp.where(vmask_flat, base, dummy)
    return gc, gs, cmask_f


def kernel(llr_ch, vn_con, cn_ids, ind_cn, ind_cn_inv):
    batch, n_vns = llr_ch.shape
    llr = -1.0 * jnp.transpose(llr_ch.astype(jnp.float32))
    gc, gs, cmask_f = _setup(vn_con, cn_ids, ind_cn, ind_cn_inv, n_vns)

    gc_chunks = gc.reshape(_SC_NW, -1, 128)
    gs_chunks = gs.reshape(_SC_NW, -1, 128)
    n_rows = N_CNS * batch // 128
    cmask_wide = jnp.broadcast_to(cmask_f[:, :, None],
                                  (DC, N_CNS, batch)).reshape(DC, n_rows, 128)

    msg_c = jnp.zeros(((DC + 1) * N_CNS, batch), jnp.float32)
    for _ in range(NUM_ITER):
        mc, _tot = _bwd_vn_fwd(msg_c, gc_chunks, gs_chunks, llr)
        msg_c = _cn_update(mc.reshape(DC, n_rows, 128),
                           cmask_wide).reshape((DC + 1) * N_CNS, batch)
    _, tot = _bwd_vn_fwd(msg_c, gc_chunks, gs_chunks, llr)
    return -1.0 * jnp.transpose(tot)


# speedup vs baseline: 2.5972x; 1.0009x over previous
import functools

import jax
import jax.numpy as jnp
from jax import lax
from jax.experimental import pallas as pl
from jax.experimental.pallas import tpu as pltpu
from jax.experimental.pallas import tpu_sc as plsc

N_CNS = 2048
DV = 3
DC = 6
NUM_ITER = 20
LLR_MAX = 20.0


def _phi(x):
    x = jnp.clip(x, 8.5e-8, 16.635532)
    t = jnp.exp(x)
    return jnp.log((t + 1.0) / (t - 1.0))


_SC_NC = 2
_SC_NS = 16
_SC_NW = _SC_NC * _SC_NS


def _bwd_vn_fwd(msg_c_flat, gc_chunks, gs_chunks, llr):
    n_vns, batch = llr.shape
    vpw = n_vns // _SC_NW
    spw = vpw * DV
    cpw = spw // 128
    mesh = plsc.VectorSubcoreMesh(core_axis_name="c", subcore_axis_name="s")

    @functools.partial(
        pl.kernel, mesh=mesh,
        out_type=[
            jax.ShapeDtypeStruct((DV * n_vns, batch), jnp.float32),
            jax.ShapeDtypeStruct((n_vns, batch), jnp.float32),
        ],
        scratch_types=[
            pltpu.VMEM((cpw, 128), jnp.int32),
            pltpu.VMEM((cpw, 128), jnp.int32),
            pltpu.VMEM((spw, batch), jnp.float32),
            pltpu.VMEM((vpw, batch), jnp.float32),
            pltpu.VMEM((spw, batch), jnp.float32),
            pltpu.VMEM((vpw, batch), jnp.float32),
            pltpu.SemaphoreType.DMA,
        ],
        compiler_params=pltpu.CompilerParams(use_tc_tiling_on_sc=False),
    )
    def bwd_vn_fwd_k(msgc_hbm, gc_hbm, gs_hbm, llr_hbm, mc_hbm, tot_hbm,
                     idx_v, idx2_v, rows_v, llr_v, out_v, tot_v, sem):
        wid = lax.axis_index("s") * _SC_NC + lax.axis_index("c")
        pltpu.sync_copy(gc_hbm.at[wid], idx_v)
        handles = [
            pltpu.async_copy(msgc_hbm.at[idx_v.at[i]],
                             rows_v.at[pl.ds(128 * i, 128)], sem)
            for i in range(cpw)
        ]
        pltpu.sync_copy(gs_hbm.at[wid], idx2_v)
        pltpu.sync_copy(llr_hbm.at[pl.ds(wid * vpw, vpw)], llr_v)
        for h in handles:
            h.wait()

        def body(vi, carry):
            base = vi * DV
            for t in range(batch // 16):
                sl = pl.ds(16 * t, 16)
                m0 = rows_v[base, sl]
                m1 = rows_v[base + 1, sl]
                m2 = rows_v[base + 2, sl]
                tt = llr_v[vi, sl] + m0 + m1 + m2
                tot_v[vi, sl] = tt
                out_v[base, sl] = tt - m0
                out_v[base + 1, sl] = tt - m1
                out_v[base + 2, sl] = tt - m2
            return carry

        lax.fori_loop(0, vpw, body, 0)
        scatters = [
            pltpu.async_copy(out_v.at[pl.ds(128 * i, 128)],
                             mc_hbm.at[idx2_v.at[i]], sem)
            for i in range(cpw)
        ]
        for h in scatters:
            h.wait()
        pltpu.sync_copy(tot_v, tot_hbm.at[pl.ds(wid * vpw, vpw)])

    return bwd_vn_fwd_k(msg_c_flat, gc_chunks, gs_chunks, llr)


def _cn_body(mc_ref, cmask_ref, out_ref):
    m = [mc_ref[k] for k in range(DC)]
    msk = [cmask_ref[k] for k in range(DC)]
    sgn = [jnp.where(msk[k] > 0.0,
                     jnp.where(m[k] < 0.0, -1.0, 1.0), 1.0) for k in range(DC)]
    mag = [jnp.where(msk[k] > 0.0,
                     _phi(jnp.clip(jnp.abs(m[k]), 0.0, LLR_MAX)), 0.0)
           for k in range(DC)]
    sign_node = sgn[0]
    mag_tot = mag[0]
    for k in range(1, DC):
        sign_node = sign_node * sgn[k]
        mag_tot = mag_tot + mag[k]
    for k in range(DC):
        out_ref[k] = (sign_node * sgn[k]) * _phi(mag_tot - mag[k])
    out_ref[DC] = jnp.zeros_like(out_ref[DC])


def _cn_update(mc, cmask_wide, *, c_blk=512):
    _, n_rows, width = mc.shape
    grid = (n_rows // c_blk,)
    return pl.pallas_call(
        _cn_body,
        grid=grid,
        in_specs=[
            pl.BlockSpec((DC, c_blk, width), lambda i: (0, i, 0)),
            pl.BlockSpec((DC, c_blk, width), lambda i: (0, i, 0)),
        ],
        out_specs=pl.BlockSpec((DC + 1, c_blk, width), lambda i: (0, i, 0)),
        out_shape=jax.ShapeDtypeStruct((DC + 1, n_rows, width), jnp.float32),
    )(mc, cmask_wide)


def _seg_slot(ids, depth):
    slot = jnp.zeros(ids.shape, jnp.int32)
    run = jnp.ones(ids.shape, jnp.bool_)
    for t in range(1, depth):
        sh = jnp.concatenate([jnp.full((t,), -1, ids.dtype), ids[:-t]])
        run = run & (ids == sh)
        slot = slot + run.astype(jnp.int32)
    return slot


def _seg_starts(ids, n_segs, num_edges):
    targets = jnp.arange(n_segs + 1, dtype=jnp.int32)
    return jnp.sum(ids.astype(jnp.int32)[None, :] < targets[:, None],
                   axis=1, dtype=jnp.int32)


def _index_compose(cs, ind_cn_inv, e_of_s_flat, n_slots):
    num_edges = cs.shape[0]
    spw = n_slots // _SC_NW
    mesh = plsc.VectorSubcoreMesh(core_axis_name="c", subcore_axis_name="s")

    @functools.partial(
        pl.kernel, mesh=mesh,
        out_type=jax.ShapeDtypeStruct((_SC_NW, spw), jnp.int32),
        scratch_types=[
            pltpu.VMEM((num_edges,), jnp.int32),
            pltpu.VMEM((num_edges,), jnp.int32),
            pltpu.VMEM((spw,), jnp.int32),
            pltpu.VMEM((spw,), jnp.int32),
        ],
        compiler_params=pltpu.CompilerParams(use_tc_tiling_on_sc=False,
                                             needs_layout_passes=False),
    )
    def compose_k(cs_hbm, inv_hbm, eos_hbm, out_hbm, cs_v, inv_v, e_v, o_v):
        wid = lax.axis_index("s") * _SC_NC + lax.axis_index("c")
        pltpu.sync_copy(cs_hbm, cs_v)
        pltpu.sync_copy(inv_hbm, inv_v)
        pltpu.sync_copy(eos_hbm.at[wid], e_v)

        def body(i, carry):
            sl = pl.ds(16 * i, 16)
            ev = e_v[sl]
            p = plsc.load_gather(inv_v, [ev])
            o_v[sl] = plsc.load_gather(cs_v, [p])
            return carry

        lax.fori_loop(0, spw // 16, body, 0)
        pltpu.sync_copy(o_v, out_hbm.at[wid])

    return compose_k(cs, ind_cn_inv,
                     e_of_s_flat.reshape(_SC_NW, spw)).reshape(-1)


def _setup(vn_con, cn_ids, ind_cn, ind_cn_inv, n_vns):
    num_edges = vn_con.shape[0]

    j_slot = _seg_slot(vn_con, DV)
    k_slot = _seg_slot(cn_ids, DC)
    cs = k_slot * N_CNS + cn_ids.astype(jnp.int32)

    vstart = _seg_starts(vn_con, n_vns, num_edges)
    cstart = _seg_starts(cn_ids, N_CNS, num_edges)
    deg_v = vstart[1:] - vstart[:-1]
    deg_c = cstart[1:] - cstart[:-1]
    vmask = (jnp.arange(DV, dtype=jnp.int32)[None, :] < deg_v[:, None])
    cmask = (jnp.arange(DC, dtype=jnp.int32)[:, None] < deg_c[None, :])

    e_of_s = jnp.minimum(vstart[:-1][:, None]
                         + jnp.arange(DV, dtype=jnp.int32)[None, :],
                         num_edges - 1)

    base = _index_compose(cs, ind_cn_inv.astype(jnp.int32),
                          e_of_s.reshape(-1), DV * n_vns)
    vmask_flat = vmask.reshape(-1)
    cmask_f = cmask.astype(jnp.float32)
    dummy = jnp.argmin(cmask_f.reshape(-1)).astype(jnp.int32)
    gc = jnp.where(vmask_flat, base, DC * N_CNS)
    gs = jnp.where(vmask_flat, base, dummy)
    return gc, gs, cmask_f


def kernel(llr_ch, vn_con, cn_ids, ind_cn, ind_cn_inv):
    batch, n_vns = llr_ch.shape
    llr = -1.0 * jnp.transpose(llr_ch.astype(jnp.float32))
    gc, gs, cmask_f = _setup(vn_con, cn_ids, ind_cn, ind_cn_inv, n_vns)

    gc_chunks = gc.reshape(_SC_NW, -1, 128)
    gs_chunks = gs.reshape(_SC_NW, -1, 128)
    n_rows = N_CNS * batch // 128
    cmask_wide = jnp.broadcast_to(cmask_f[:, :, None],
                                  (DC, N_CNS, batch)).reshape(DC, n_rows, 128)

    msg_c = jnp.zeros(((DC + 1) * N_CNS, batch), jnp.float32)
    for _ in range(NUM_ITER):
        mc, _tot = _bwd_vn_fwd(msg_c, gc_chunks, gs_chunks, llr)
        msg_c = _cn_update(mc.reshape(DC, n_rows, 128),
                           cmask_wide).reshape((DC + 1) * N_CNS, batch)
    _, tot = _bwd_vn_fwd(msg_c, gc_chunks, gs_chunks, llr)
    return -1.0 * jnp.transpose(tot)
